# Initial kernel scaffold; baseline (speedup 1.0000x reference)
import functools

import jax
import jax.numpy as jnp
from jax import lax
from jax.experimental import pallas as pl
from jax.experimental.pallas import tpu as pltpu
from jax.experimental.pallas import tpu_sc as plsc

N = 10000
E = 320000
D = 128

NC = 2
NS = 16
NW = NC * NS

CH = 128
EPT = 10112
NCHUNK = EPT // CH
E_PAD = NW * EPT
ACC_ROWS = 10240
ROWS_PER_TILE = N // NS
ZCH = 128

BN = 1000
NBLK = N // BN

_mesh = plsc.VectorSubcoreMesh(core_axis_name="c", subcore_axis_name="s")


def _deg_body(dst_hbm, out_hbm, dst_v, hist_v, ones_v):
    c = lax.axis_index("c")
    s = lax.axis_index("s")
    wid = c * NS + s

    def zero_hist(i, _):
        hist_v[pl.ds(i * 16, 16)] = jnp.zeros((16,), jnp.float32)
        return 0

    lax.fori_loop(0, ACC_ROWS // 16, zero_hist, 0)
    ones_v[...] = jnp.ones((16,), jnp.float32)

    def chunk(j, _):
        base = wid * EPT + j * CH
        pltpu.sync_copy(dst_hbm.at[pl.ds(base, CH)], dst_v)

        def lane(i, _):
            idx = dst_v[pl.ds(i * 16, 16)]
            plsc.addupdate_scatter(hist_v, [idx], ones_v[...])
            return 0

        lax.fori_loop(0, CH // 16, lane, 0)
        return 0

    lax.fori_loop(0, NCHUNK, chunk, 0)
    pltpu.sync_copy(hist_v, out_hbm.at[wid])


_deg_kernel = functools.partial(
    pl.kernel,
    out_type=jax.ShapeDtypeStruct((NW, ACC_ROWS), jnp.float32),
    mesh=_mesh,
    scratch_types=[
        pltpu.VMEM((CH,), jnp.int32),
        pltpu.VMEM((ACC_ROWS,), jnp.float32),
        pltpu.VMEM((16,), jnp.float32),
    ],
)(_deg_body)


def _scatter_body(src_hbm, dst_hbm, m_hbm, out_hbm,
                  acc_sh, src_v, dst_v, rows_v, sem):
    c = lax.axis_index("c")
    s = lax.axis_index("s")
    wid = c * NS + s

    def zrow(i, _):
        def zlane(l, _):
            rows_v[i, pl.ds(l * 16, 16)] = jnp.zeros((16,), jnp.float32)
            return 0
        lax.fori_loop(0, D // 16, zlane, 0)
        return 0

    lax.fori_loop(0, ZCH, zrow, 0)

    def zcopy(i, _):
        pltpu.sync_copy(
            rows_v,
            acc_sh.at[pl.ds((s * (ACC_ROWS // NS)) + i * ZCH, ZCH)])
        return 0

    lax.fori_loop(0, (ACC_ROWS // NS) // ZCH, zcopy, 0)
    plsc.subcore_barrier()

    def chunk(j, _):
        base = wid * EPT + j * CH
        pltpu.sync_copy(src_hbm.at[pl.ds(base, CH)], src_v)
        pltpu.sync_copy(dst_hbm.at[pl.ds(base, CH)], dst_v)
        pltpu.async_copy(m_hbm.at[src_v], rows_v, sem).wait()
        pltpu.sync_copy(rows_v, acc_sh.at[dst_v], add=True)
        return 0

    lax.fori_loop(0, NCHUNK, chunk, 0)
    plsc.subcore_barrier()
    pltpu.sync_copy(acc_sh.at[pl.ds(s * ROWS_PER_TILE, ROWS_PER_TILE)],
                    out_hbm.at[c, pl.ds(s * ROWS_PER_TILE, ROWS_PER_TILE)])


_scatter_kernel = functools.partial(
    pl.kernel,
    out_type=jax.ShapeDtypeStruct((NC, N, D), jnp.float32),
    mesh=_mesh,
    scratch_types=[
        pltpu.VMEM_SHARED((ACC_ROWS, D), jnp.float32),
        pltpu.VMEM((CH,), jnp.int32),
        pltpu.VMEM((CH,), jnp.int32),
        pltpu.VMEM((ZCH, D), jnp.float32),
        pltpu.SemaphoreType.DMA,
    ],
)(_scatter_body)


def _prep_body(deg_ref, x_ref, w_ref, m_ref, dinv_ref):
    deg = jnp.sum(deg_ref[...], axis=0) + 1.0
    dinv = lax.rsqrt(deg)
    h = jnp.dot(x_ref[...], w_ref[...], preferred_element_type=jnp.float32)
    m_ref[...] = h * dinv[:, None]
    dinv_ref[...] = dinv[:, None]


_prep_kernel = pl.pallas_call(
    _prep_body,
    grid=(NBLK,),
    in_specs=[
        pl.BlockSpec((NW, BN), lambda i: (0, i)),
        pl.BlockSpec((BN, D), lambda i: (i, 0)),
        pl.BlockSpec((D, D), lambda i: (0, 0)),
    ],
    out_specs=[
        pl.BlockSpec((BN, D), lambda i: (i, 0)),
        pl.BlockSpec((BN, 1), lambda i: (i, 0)),
    ],
    out_shape=[
        jax.ShapeDtypeStruct((N, D), jnp.float32),
        jax.ShapeDtypeStruct((N, 1), jnp.float32),
    ],
)


def _combine_body(p_ref, m_ref, dinv_ref, b_ref, w_ref, m2_ref):
    y = (p_ref[0] + p_ref[1] + m_ref[...]) * dinv_ref[...] + b_ref[...]
    r = jnp.maximum(y, 0.0)
    h = jnp.dot(r, w_ref[...], preferred_element_type=jnp.float32)
    m2_ref[...] = h * dinv_ref[...]


_combine_kernel = pl.pallas_call(
    _combine_body,
    grid=(NBLK,),
    in_specs=[
        pl.BlockSpec((NC, BN, D), lambda i: (0, i, 0)),
        pl.BlockSpec((BN, D), lambda i: (i, 0)),
        pl.BlockSpec((BN, 1), lambda i: (i, 0)),
        pl.BlockSpec((1, D), lambda i: (0, 0)),
        pl.BlockSpec((D, D), lambda i: (0, 0)),
    ],
    out_specs=pl.BlockSpec((BN, D), lambda i: (i, 0)),
    out_shape=jax.ShapeDtypeStruct((N, D), jnp.float32),
)


def _final_body(p_ref, m_ref, dinv_ref, b_ref, wg1_ref, bg1_ref,
                wg2_ref, bg2_ref, gamma_ref, beta_ref, out_ref, acc_ref):
    i = pl.program_id(0)
    y = (p_ref[0] + p_ref[1] + m_ref[...]) * dinv_ref[...] + b_ref[...]
    part = jnp.sum(y.reshape(BN // 8, 8, D), axis=0)

    @pl.when(i == 0)
    def _():
        acc_ref[...] = jnp.zeros_like(acc_ref)

    acc_ref[...] += part

    @pl.when(i == NBLK - 1)
    def _():
        g = jnp.sum(acc_ref[...], axis=0, keepdims=True) * (1.0 / N)
        z = jnp.maximum(
            jnp.dot(g, wg1_ref[...], preferred_element_type=jnp.float32)
            + bg1_ref[...], 0.0)
        z = jnp.dot(z, wg2_ref[...], preferred_element_type=jnp.float32) \
            + bg2_ref[...]
        mu = jnp.mean(z, axis=-1, keepdims=True)
        zc = z - mu
        var = jnp.mean(zc * zc, axis=-1, keepdims=True)
        out_ref[...] = zc * lax.rsqrt(var + 1e-5) * gamma_ref[...] \
            + beta_ref[...]


_final_kernel = pl.pallas_call(
    _final_body,
    grid=(NBLK,),
    in_specs=[
        pl.BlockSpec((NC, BN, D), lambda i: (0, i, 0)),
        pl.BlockSpec((BN, D), lambda i: (i, 0)),
        pl.BlockSpec((BN, 1), lambda i: (i, 0)),
        pl.BlockSpec((1, D), lambda i: (0, 0)),
        pl.BlockSpec((D, D), lambda i: (0, 0)),
        pl.BlockSpec((1, D), lambda i: (0, 0)),
        pl.BlockSpec((D, D), lambda i: (0, 0)),
        pl.BlockSpec((1, D), lambda i: (0, 0)),
        pl.BlockSpec((1, D), lambda i: (0, 0)),
        pl.BlockSpec((1, D), lambda i: (0, 0)),
    ],
    out_specs=pl.BlockSpec((1, D), lambda i: (0, 0)),
    out_shape=jax.ShapeDtypeStruct((1, D), jnp.float32),
    scratch_shapes=[pltpu.VMEM((8, D), jnp.float32)],
)


def kernel(node_features, edge_index, W1, b1, W2, b2, W3, b3,
           Wg1, bg1, Wg2, bg2, gamma, beta):
    src = jnp.concatenate(
        [edge_index[0], jnp.zeros((E_PAD - E,), jnp.int32)])
    dst = jnp.concatenate(
        [edge_index[1], jnp.full((E_PAD - E,), N, jnp.int32)])

    deg32 = _deg_kernel(dst)[:, :N]

    m1, dinv = _prep_kernel(deg32, node_features, W1)
    p1 = _scatter_kernel(src, dst, m1)
    m2 = _combine_kernel(p1, m1, dinv, b1.reshape(1, D), W2)
    p2 = _scatter_kernel(src, dst, m2)
    m3 = _combine_kernel(p2, m2, dinv, b2.reshape(1, D), W3)
    p3 = _scatter_kernel(src, dst, m3)
    z = _final_kernel(p3, m3, dinv, b3.reshape(1, D),
                      Wg1, bg1.reshape(1, D), Wg2, bg2.reshape(1, D),
                      gamma.reshape(1, D), beta.reshape(1, D))
    return z


# speedup vs baseline: 9.1744x; 9.1744x over previous
import functools

import jax
import jax.numpy as jnp
from jax import lax
from jax.experimental import pallas as pl
from jax.experimental.pallas import tpu as pltpu
from jax.experimental.pallas import tpu_sc as plsc

N = 10000
E = 320000
D = 128

NC = 2
NS = 16
NW = NC * NS

CH = 128
EPT = 10112
NCHUNK = EPT // CH
E_PAD = NW * EPT
ACC_ROWS = 10240
ROWS_PER_TILE = ACC_ROWS // NS
ZCH = 128

BN = 1000
NBLK = N // BN

_mesh = plsc.VectorSubcoreMesh(core_axis_name="c", subcore_axis_name="s")


def _deg_body(dst_hbm, out_hbm, dst_v, hist_v, ones_v):
    c = lax.axis_index("c")
    s = lax.axis_index("s")
    wid = c * NS + s

    def zero_hist(i, _):
        hist_v[pl.ds(i * 16, 16)] = jnp.zeros((16,), jnp.float32)
        return 0

    lax.fori_loop(0, ACC_ROWS // 16, zero_hist, 0)
    ones_v[...] = jnp.ones((16,), jnp.float32)

    def chunk(j, _):
        base = wid * EPT + j * CH
        pltpu.sync_copy(dst_hbm.at[pl.ds(base, CH)], dst_v)

        def lane(i, _):
            idx = dst_v[pl.ds(i * 16, 16)]
            plsc.addupdate_scatter(hist_v, [idx], ones_v[...])
            return 0

        lax.fori_loop(0, CH // 16, lane, 0)
        return 0

    lax.fori_loop(0, NCHUNK, chunk, 0)
    pltpu.sync_copy(hist_v, out_hbm.at[wid])


_deg_kernel = functools.partial(
    pl.kernel,
    out_type=jax.ShapeDtypeStruct((NW, ACC_ROWS), jnp.float32),
    mesh=_mesh,
    compiler_params=pltpu.CompilerParams(needs_layout_passes=False),
    scratch_types=[
        pltpu.VMEM((CH,), jnp.int32),
        pltpu.VMEM((ACC_ROWS,), jnp.float32),
        pltpu.VMEM((16,), jnp.float32),
    ],
)(_deg_body)


def _scatter_body(src_hbm, dst_hbm, m_hbm, out_hbm,
                  acc_sh, src_v, dst_v, rows_v, sem):
    c = lax.axis_index("c")
    s = lax.axis_index("s")
    wid = c * NS + s

    def zrow(i, _):
        def zlane(l, _):
            rows_v[i, pl.ds(l * 16, 16)] = jnp.zeros((16,), jnp.float32)
            return 0
        lax.fori_loop(0, D // 16, zlane, 0)
        return 0

    lax.fori_loop(0, ZCH, zrow, 0)

    def zcopy(i, _):
        pltpu.sync_copy(
            rows_v,
            acc_sh.at[pl.ds((s * (ACC_ROWS // NS)) + i * ZCH, ZCH)])
        return 0

    lax.fori_loop(0, (ACC_ROWS // NS) // ZCH, zcopy, 0)
    plsc.subcore_barrier()

    def chunk(j, _):
        base = wid * EPT + j * CH
        pltpu.sync_copy(src_hbm.at[pl.ds(base, CH)], src_v)
        pltpu.sync_copy(dst_hbm.at[pl.ds(base, CH)], dst_v)
        pltpu.async_copy(m_hbm.at[src_v], rows_v, sem).wait()
        pltpu.sync_copy(rows_v, acc_sh.at[dst_v], add=True)
        return 0

    lax.fori_loop(0, NCHUNK, chunk, 0)
    plsc.subcore_barrier()
    pltpu.sync_copy(acc_sh.at[pl.ds(s * ROWS_PER_TILE, ROWS_PER_TILE)],
                    out_hbm.at[c, pl.ds(s * ROWS_PER_TILE, ROWS_PER_TILE)])


_scatter_kernel = functools.partial(
    pl.kernel,
    out_type=jax.ShapeDtypeStruct((NC, ACC_ROWS, D), jnp.float32),
    mesh=_mesh,
    compiler_params=pltpu.CompilerParams(needs_layout_passes=False),
    scratch_types=[
        pltpu.VMEM_SHARED((ACC_ROWS, D), jnp.float32),
        pltpu.VMEM((CH,), jnp.int32),
        pltpu.VMEM((CH,), jnp.int32),
        pltpu.VMEM((ZCH, D), jnp.float32),
        pltpu.SemaphoreType.DMA,
    ],
)(_scatter_body)


def _prep_body(deg_ref, x_ref, w_ref, m_ref, dinv_ref):
    deg = jnp.sum(deg_ref[...], axis=1) + 1.0
    dinv = lax.rsqrt(deg)
    h = jnp.dot(x_ref[...], w_ref[...], preferred_element_type=jnp.float32)
    m_ref[...] = h * dinv[:, None]
    dinv_ref[...] = dinv[:, None]


_prep_kernel = pl.pallas_call(
    _prep_body,
    grid=(NBLK,),
    in_specs=[
        pl.BlockSpec((BN, NW), lambda i: (i, 0)),
        pl.BlockSpec((BN, D), lambda i: (i, 0)),
        pl.BlockSpec((D, D), lambda i: (0, 0)),
    ],
    out_specs=[
        pl.BlockSpec((BN, D), lambda i: (i, 0)),
        pl.BlockSpec((BN, 1), lambda i: (i, 0)),
    ],
    out_shape=[
        jax.ShapeDtypeStruct((N, D), jnp.float32),
        jax.ShapeDtypeStruct((N, 1), jnp.float32),
    ],
)


def _combine_body(p_ref, m_ref, dinv_ref, b_ref, w_ref, m2_ref):
    y = (p_ref[0] + p_ref[1] + m_ref[...]) * dinv_ref[...] + b_ref[...]
    r = jnp.maximum(y, 0.0)
    h = jnp.dot(r, w_ref[...], preferred_element_type=jnp.float32)
    m2_ref[...] = h * dinv_ref[...]


_combine_kernel = pl.pallas_call(
    _combine_body,
    grid=(NBLK,),
    in_specs=[
        pl.BlockSpec((NC, BN, D), lambda i: (0, i, 0)),
        pl.BlockSpec((BN, D), lambda i: (i, 0)),
        pl.BlockSpec((BN, 1), lambda i: (i, 0)),
        pl.BlockSpec((1, D), lambda i: (0, 0)),
        pl.BlockSpec((D, D), lambda i: (0, 0)),
    ],
    out_specs=pl.BlockSpec((BN, D), lambda i: (i, 0)),
    out_shape=jax.ShapeDtypeStruct((N, D), jnp.float32),
)


def _final_body(p_ref, m_ref, dinv_ref, b_ref, wg1_ref, bg1_ref,
                wg2_ref, bg2_ref, gamma_ref, beta_ref, out_ref, acc_ref):
    i = pl.program_id(0)
    y = (p_ref[0] + p_ref[1] + m_ref[...]) * dinv_ref[...] + b_ref[...]
    part = jnp.sum(y.reshape(BN // 8, 8, D), axis=0)

    @pl.when(i == 0)
    def _():
        acc_ref[...] = jnp.zeros_like(acc_ref)

    acc_ref[...] += part

    @pl.when(i == NBLK - 1)
    def _():
        g = jnp.sum(acc_ref[...], axis=0, keepdims=True) * (1.0 / N)
        z = jnp.maximum(
            jnp.dot(g, wg1_ref[...], preferred_element_type=jnp.float32)
            + bg1_ref[...], 0.0)
        z = jnp.dot(z, wg2_ref[...], preferred_element_type=jnp.float32) \
            + bg2_ref[...]
        mu = jnp.mean(z, axis=-1, keepdims=True)
        zc = z - mu
        var = jnp.mean(zc * zc, axis=-1, keepdims=True)
        out_ref[...] = zc * lax.rsqrt(var + 1e-5) * gamma_ref[...] \
            + beta_ref[...]


_final_kernel = pl.pallas_call(
    _final_body,
    grid=(NBLK,),
    in_specs=[
        pl.BlockSpec((NC, BN, D), lambda i: (0, i, 0)),
        pl.BlockSpec((BN, D), lambda i: (i, 0)),
        pl.BlockSpec((BN, 1), lambda i: (i, 0)),
        pl.BlockSpec((1, D), lambda i: (0, 0)),
        pl.BlockSpec((D, D), lambda i: (0, 0)),
        pl.BlockSpec((1, D), lambda i: (0, 0)),
        pl.BlockSpec((D, D), lambda i: (0, 0)),
        pl.BlockSpec((1, D), lambda i: (0, 0)),
        pl.BlockSpec((1, D), lambda i: (0, 0)),
        pl.BlockSpec((1, D), lambda i: (0, 0)),
    ],
    out_specs=pl.BlockSpec((1, D), lambda i: (0, 0)),
    out_shape=jax.ShapeDtypeStruct((1, D), jnp.float32),
    scratch_shapes=[pltpu.VMEM((8, D), jnp.float32)],
)


def kernel(node_features, edge_index, W1, b1, W2, b2, W3, b3,
           Wg1, bg1, Wg2, bg2, gamma, beta):
    src = jnp.concatenate(
        [edge_index[0], jnp.zeros((E_PAD - E,), jnp.int32)])
    dst = jnp.concatenate(
        [edge_index[1], jnp.full((E_PAD - E,), N, jnp.int32)])

    deg_t = _deg_kernel(dst).T[:N]

    m1, dinv = _prep_kernel(deg_t, node_features, W1)
    p1 = _scatter_kernel(src, dst, m1)[:, :N]
    m2 = _combine_kernel(p1, m1, dinv, b1.reshape(1, D), W2)
    p2 = _scatter_kernel(src, dst, m2)[:, :N]
    m3 = _combine_kernel(p2, m2, dinv, b2.reshape(1, D), W3)
    p3 = _scatter_kernel(src, dst, m3)[:, :N]
    z = _final_kernel(p3, m3, dinv, b3.reshape(1, D),
                      Wg1, bg1.reshape(1, D), Wg2, bg2.reshape(1, D),
                      gamma.reshape(1, D), beta.reshape(1, D))
    return z
